# baseline (speedup 1.0000x reference)
import functools

import jax
import jax.numpy as jnp
from jax import lax
from jax.experimental import pallas as pl
from jax.experimental.pallas import tpu as pltpu
from jax.experimental.pallas import tpu_sc as plsc

_NUM_CODES = 8192
_EMBED_DIM = 1024
_M_BLK = 256
_C_BLK = 512


def _argmax_body(x_ref, w_ref, idx_ref, loss_ref,
                 m_scr, a_scr, wn_scr, u_scr, acc_scr):
    i = pl.program_id(0)
    j = pl.program_id(1)
    nj = pl.num_programs(1)

    x = x_ref[...]
    xsq = jnp.sum(x * x, axis=1, keepdims=True)
    xn = jnp.sqrt(xsq)
    xhat = x / jnp.maximum(xn, 1e-12)
    w = w_ref[...]
    wn = jnp.sqrt(jnp.sum(w * w, axis=1, keepdims=True))
    what = w / jnp.maximum(wn, 1e-12)

    s = lax.dot_general(xhat, what, (((1,), (1,)), ((), ())),
                        preferred_element_type=jnp.float32)
    col_i = jax.lax.broadcasted_iota(jnp.int32, s.shape, 1)
    col_f = col_i.astype(jnp.float32)
    wn_row = wn[:, 0][None, :]

    def seg_stats(mask):
        sm = jnp.where(mask, s, -jnp.inf)
        m = jnp.max(sm, axis=1, keepdims=True)
        argf = jnp.min(jnp.where(sm == m, col_f, 1e9), axis=1, keepdims=True)
        arg = argf.astype(jnp.int32)
        oh = col_i == arg
        wnb = jnp.sum(jnp.where(oh, wn_row, 0.0), axis=1, keepdims=True)
        ub = jnp.sum(jnp.where(oh, s, 0.0), axis=1, keepdims=True)
        return m, arg + j * _C_BLK, wnb, ub

    def merge(m, a, wnb, ub):
        better = m > m_scr[...]
        a_scr[...] = jnp.where(better, a, a_scr[...])
        wn_scr[...] = jnp.where(better, wnb, wn_scr[...])
        u_scr[...] = jnp.where(better, ub, u_scr[...])
        m_scr[...] = jnp.where(better, m, m_scr[...])

    boundary = (j == 5) | (j == 10)
    off = jnp.where(j == 5, 176, jnp.where(j == 10, 352, _C_BLK))
    mA, aA, wA, uA = seg_stats(col_i < off)

    @pl.when(j == 0)
    def _():
        m_scr[...] = mA
        a_scr[...] = aA
        wn_scr[...] = wA
        u_scr[...] = uA

    @pl.when(j > 0)
    def _():
        merge(mA, aA, wA, uA)

    @pl.when(boundary)
    def _():
        m_scr[...] = m_scr[...].astype(jnp.bfloat16).astype(jnp.float32)
        mB, aB, wB, uB = seg_stats(col_i >= off)
        merge(mB, aB, wB, uB)

    @pl.when(j == nj - 1)
    def _():
        idx_ref[0, 0, :] = a_scr[...][:, 0]
        best_wn = wn_scr[...]
        lrow = xsq - 2.0 * (u_scr[...] * xn * best_wn) + best_wn * best_wn

        @pl.when(i == 0)
        def _():
            acc_scr[0, 0] = 0.0

        acc_scr[0, 0] += jnp.sum(lrow)
        loss_ref[...] = jnp.full(
            (1, 1), acc_scr[0, 0] * (1.25 / (16384.0 * 1024.0)), jnp.float32)


def _matmul_argmax(x, W):
    m, d = x.shape
    c = W.shape[0]
    ni, nj = m // _M_BLK, c // _C_BLK
    return pl.pallas_call(
        _argmax_body,
        grid=(ni, nj),
        in_specs=[
            pl.BlockSpec((_M_BLK, d), lambda i, j: (i, 0)),
            pl.BlockSpec((_C_BLK, d), lambda i, j: (j, 0)),
        ],
        out_specs=[
            pl.BlockSpec((1, 1, _M_BLK), lambda i, j: (i, 0, 0)),
            pl.BlockSpec((1, 1), lambda i, j: (0, 0)),
        ],
        out_shape=[
            jax.ShapeDtypeStruct((ni, 1, _M_BLK), jnp.int32),
            jax.ShapeDtypeStruct((1, 1), jnp.float32),
        ],
        scratch_shapes=[
            pltpu.VMEM((_M_BLK, 1), jnp.float32),
            pltpu.VMEM((_M_BLK, 1), jnp.int32),
            pltpu.VMEM((_M_BLK, 1), jnp.float32),
            pltpu.VMEM((_M_BLK, 1), jnp.float32),
            pltpu.SMEM((1, 1), jnp.float32),
        ],
    )(x, W)


def _sc_gather_hist(W, idx):
    n = idx.shape[0]
    d = W.shape[1]
    info = plsc.get_sparse_core_info()
    nw = info.num_cores * info.num_subcores
    per_w = n // nw
    chunk = 32
    mesh = plsc.VectorSubcoreMesh(core_axis_name="c", subcore_axis_name="s")

    @functools.partial(
        pl.kernel,
        out_type=(jax.ShapeDtypeStruct((n, d), jnp.float32),
                  jax.ShapeDtypeStruct((nw, _NUM_CODES), jnp.float32)),
        mesh=mesh,
        compiler_params=pltpu.CompilerParams(needs_layout_passes=False),
        scratch_types=[
            pltpu.VMEM((per_w,), jnp.int32),
            pltpu.VMEM((chunk, d), jnp.float32),
            pltpu.VMEM((_NUM_CODES,), jnp.float32),
            pltpu.SemaphoreType.DMA,
        ],
    )
    def k(w_hbm, idx_hbm, out_hbm, cnt_hbm, idx_v, rows_v, cnt_v, sem):
        wid = lax.axis_index("s") * info.num_cores + lax.axis_index("c")
        base = wid * per_w
        pltpu.sync_copy(idx_hbm.at[pl.ds(base, per_w)], idx_v)

        zeros16 = jnp.zeros((16,), jnp.float32)

        def zbody(t, carry):
            cnt_v[pl.ds(t * 16, 16)] = zeros16
            return carry

        lax.fori_loop(0, _NUM_CODES // 16, zbody, 0)

        ones16 = jnp.ones((16,), jnp.float32)

        def hbody(t, carry):
            v = idx_v[pl.ds(t * 16, 16)]
            plsc.addupdate_scatter(cnt_v, [v], ones16)
            return carry

        lax.fori_loop(0, per_w // 16, hbody, 0)
        pltpu.sync_copy(cnt_v, cnt_hbm.at[wid])

        def gbody(g, carry):
            pltpu.async_copy(w_hbm.at[idx_v.at[pl.ds(g * chunk, chunk)]],
                             rows_v, sem).wait()
            pltpu.sync_copy(rows_v, out_hbm.at[pl.ds(base + g * chunk, chunk)])
            return carry

        lax.fori_loop(0, per_w // chunk, gbody, 0)

    return k(W, idx)


def _perp_body(cnt_ref, perp_ref):
    counts = jnp.sum(cnt_ref[...], axis=0, keepdims=True)
    probs = counts * (1.0 / 16384.0)
    ent = -jnp.sum(probs * jnp.log(probs + 1e-10))
    perp_ref[...] = jnp.full((1, 1), jnp.exp(ent), jnp.float32)


def _perplexity(cnt):
    return pl.pallas_call(
        _perp_body,
        out_shape=jax.ShapeDtypeStruct((1, 1), jnp.float32),
    )(cnt)


def kernel(inputs, W):
    b, npatch, d = inputs.shape
    x = inputs.reshape(-1, d)
    idx3, loss = _matmul_argmax(x, W)
    idx = idx3.reshape(-1)
    quant, cnt = _sc_gather_hist(W, idx)
    perp = _perplexity(cnt)
    return (quant.reshape(b, npatch, d), loss.reshape(()), perp.reshape(()))


# speedup vs baseline: 1.0367x; 1.0367x over previous
import functools

import jax
import jax.numpy as jnp
from jax import lax
from jax.experimental import pallas as pl
from jax.experimental.pallas import tpu as pltpu
from jax.experimental.pallas import tpu_sc as plsc

_NUM_CODES = 8192
_EMBED_DIM = 1024
_M_BLK = 256
_C_BLK = 512
_NORM_BLK = 1024


def _norm_x_body(x_ref, out_ref):
    x = x_ref[...]
    n = jnp.sqrt(jnp.sum(x * x, axis=1, keepdims=True))
    out_ref[...] = x / jnp.maximum(n, 1e-12)


def _norm_w_body(w_ref, out_ref, wn_ref):
    w = w_ref[...]
    n = jnp.sqrt(jnp.sum(w * w, axis=1, keepdims=True))
    out_ref[...] = w / jnp.maximum(n, 1e-12)
    wn_ref[0, 0, :] = n[:, 0]


def _normalize_x(x):
    m, d = x.shape
    return pl.pallas_call(
        _norm_x_body,
        grid=(m // _NORM_BLK,),
        in_specs=[pl.BlockSpec((_NORM_BLK, d), lambda i: (i, 0))],
        out_specs=pl.BlockSpec((_NORM_BLK, d), lambda i: (i, 0)),
        out_shape=jax.ShapeDtypeStruct((m, d), jnp.float32),
    )(x)


def _normalize_w(W):
    c, d = W.shape
    nb = c // _NORM_BLK
    return pl.pallas_call(
        _norm_w_body,
        grid=(nb,),
        in_specs=[pl.BlockSpec((_NORM_BLK, d), lambda i: (i, 0))],
        out_specs=[
            pl.BlockSpec((_NORM_BLK, d), lambda i: (i, 0)),
            pl.BlockSpec((1, 1, _NORM_BLK), lambda i: (i, 0, 0)),
        ],
        out_shape=[
            jax.ShapeDtypeStruct((c, d), jnp.float32),
            jax.ShapeDtypeStruct((nb, 1, _NORM_BLK), jnp.float32),
        ],
    )(W)


def _argmax_body(x_ref, xhat_ref, what_ref, wn_ref, idx_ref, loss_ref,
                 m_scr, a_scr, wn_scr, u_scr, acc_scr):
    i = pl.program_id(0)
    j = pl.program_id(1)
    nj = pl.num_programs(1)

    xhat = xhat_ref[...]
    what = what_ref[...]
    wn_row = wn_ref[0, 0, :][None, :]

    s = lax.dot_general(xhat, what, (((1,), (1,)), ((), ())),
                        preferred_element_type=jnp.float32)
    col_i = jax.lax.broadcasted_iota(jnp.int32, s.shape, 1)
    col_f = col_i.astype(jnp.float32)

    def seg_stats(mask):
        sm = jnp.where(mask, s, -jnp.inf)
        m = jnp.max(sm, axis=1, keepdims=True)
        argf = jnp.min(jnp.where(sm == m, col_f, 1e9), axis=1, keepdims=True)
        arg = argf.astype(jnp.int32)
        oh = col_i == arg
        wnb = jnp.sum(jnp.where(oh, wn_row, 0.0), axis=1, keepdims=True)
        ub = jnp.sum(jnp.where(oh, s, 0.0), axis=1, keepdims=True)
        return m, arg + j * _C_BLK, wnb, ub

    def merge(m, a, wnb, ub):
        better = m > m_scr[...]
        a_scr[...] = jnp.where(better, a, a_scr[...])
        wn_scr[...] = jnp.where(better, wnb, wn_scr[...])
        u_scr[...] = jnp.where(better, ub, u_scr[...])
        m_scr[...] = jnp.where(better, m, m_scr[...])

    boundary = (j == 5) | (j == 10)
    off = jnp.where(j == 5, 176, jnp.where(j == 10, 352, _C_BLK))
    mA, aA, wA, uA = seg_stats(col_i < off)

    @pl.when(j == 0)
    def _():
        m_scr[...] = mA
        a_scr[...] = aA
        wn_scr[...] = wA
        u_scr[...] = uA

    @pl.when(j > 0)
    def _():
        merge(mA, aA, wA, uA)

    @pl.when(boundary)
    def _():
        m_scr[...] = m_scr[...].astype(jnp.bfloat16).astype(jnp.float32)
        mB, aB, wB, uB = seg_stats(col_i >= off)
        merge(mB, aB, wB, uB)

    @pl.when(j == nj - 1)
    def _():
        idx_ref[0, 0, :] = a_scr[...][:, 0]
        x = x_ref[...]
        xsq = jnp.sum(x * x, axis=1, keepdims=True)
        xn = jnp.sqrt(xsq)
        best_wn = wn_scr[...]
        lrow = xsq - 2.0 * (u_scr[...] * xn * best_wn) + best_wn * best_wn

        @pl.when(i == 0)
        def _():
            acc_scr[0, 0] = 0.0

        acc_scr[0, 0] += jnp.sum(lrow)
        loss_ref[...] = jnp.full(
            (1, 1), acc_scr[0, 0] * (1.25 / (16384.0 * 1024.0)), jnp.float32)


def _matmul_argmax(x, xhat, what, wn3):
    m, d = x.shape
    c = what.shape[0]
    ni, nj = m // _M_BLK, c // _C_BLK
    return pl.pallas_call(
        _argmax_body,
        grid=(ni, nj),
        in_specs=[
            pl.BlockSpec((_M_BLK, d), lambda i, j: (i, 0)),
            pl.BlockSpec((_M_BLK, d), lambda i, j: (i, 0)),
            pl.BlockSpec((_C_BLK, d), lambda i, j: (j, 0)),
            pl.BlockSpec((1, 1, _C_BLK),
                         lambda i, j: (j * _C_BLK // _NORM_BLK, 0,
                                       (j * _C_BLK % _NORM_BLK) // _C_BLK)),
        ],
        out_specs=[
            pl.BlockSpec((1, 1, _M_BLK), lambda i, j: (i, 0, 0)),
            pl.BlockSpec((1, 1), lambda i, j: (0, 0)),
        ],
        out_shape=[
            jax.ShapeDtypeStruct((ni, 1, _M_BLK), jnp.int32),
            jax.ShapeDtypeStruct((1, 1), jnp.float32),
        ],
        scratch_shapes=[
            pltpu.VMEM((_M_BLK, 1), jnp.float32),
            pltpu.VMEM((_M_BLK, 1), jnp.int32),
            pltpu.VMEM((_M_BLK, 1), jnp.float32),
            pltpu.VMEM((_M_BLK, 1), jnp.float32),
            pltpu.SMEM((1, 1), jnp.float32),
        ],
    )(x, xhat, what, wn3)


def _sc_gather_hist(W, idx):
    n = idx.shape[0]
    d = W.shape[1]
    info = plsc.get_sparse_core_info()
    nw = info.num_cores * info.num_subcores
    per_w = n // nw
    chunk = 32
    mesh = plsc.VectorSubcoreMesh(core_axis_name="c", subcore_axis_name="s")

    @functools.partial(
        pl.kernel,
        out_type=(jax.ShapeDtypeStruct((n, d), jnp.float32),
                  jax.ShapeDtypeStruct((nw, _NUM_CODES), jnp.float32)),
        mesh=mesh,
        compiler_params=pltpu.CompilerParams(needs_layout_passes=False),
        scratch_types=[
            pltpu.VMEM((per_w,), jnp.int32),
            pltpu.VMEM((chunk, d), jnp.float32),
            pltpu.VMEM((_NUM_CODES,), jnp.float32),
            pltpu.SemaphoreType.DMA,
        ],
    )
    def k(w_hbm, idx_hbm, out_hbm, cnt_hbm, idx_v, rows_v, cnt_v, sem):
        wid = lax.axis_index("s") * info.num_cores + lax.axis_index("c")
        base = wid * per_w
        pltpu.sync_copy(idx_hbm.at[pl.ds(base, per_w)], idx_v)

        zeros16 = jnp.zeros((16,), jnp.float32)

        def zbody(t, carry):
            cnt_v[pl.ds(t * 16, 16)] = zeros16
            return carry

        lax.fori_loop(0, _NUM_CODES // 16, zbody, 0)

        ones16 = jnp.ones((16,), jnp.float32)

        def hbody(t, carry):
            v = idx_v[pl.ds(t * 16, 16)]
            plsc.addupdate_scatter(cnt_v, [v], ones16)
            return carry

        lax.fori_loop(0, per_w // 16, hbody, 0)
        pltpu.sync_copy(cnt_v, cnt_hbm.at[wid])

        def gbody(g, carry):
            pltpu.async_copy(w_hbm.at[idx_v.at[pl.ds(g * chunk, chunk)]],
                             rows_v, sem).wait()
            pltpu.sync_copy(rows_v, out_hbm.at[pl.ds(base + g * chunk, chunk)])
            return carry

        lax.fori_loop(0, per_w // chunk, gbody, 0)

    return k(W, idx)


def _perp_body(cnt_ref, perp_ref):
    counts = jnp.sum(cnt_ref[...], axis=0, keepdims=True)
    probs = counts * (1.0 / 16384.0)
    ent = -jnp.sum(probs * jnp.log(probs + 1e-10))
    perp_ref[...] = jnp.full((1, 1), jnp.exp(ent), jnp.float32)


def _perplexity(cnt):
    return pl.pallas_call(
        _perp_body,
        out_shape=jax.ShapeDtypeStruct((1, 1), jnp.float32),
    )(cnt)


def kernel(inputs, W):
    b, npatch, d = inputs.shape
    x = inputs.reshape(-1, d)
    xhat = _normalize_x(x)
    what, wn3 = _normalize_w(W)
    idx3, loss = _matmul_argmax(x, xhat, what, wn3)
    idx = idx3.reshape(-1)
    quant, cnt = _sc_gather_hist(W, idx)
    perp = _perplexity(cnt)
    return (quant.reshape(b, npatch, d), loss.reshape(()), perp.reshape(()))


# speedup vs baseline: 1.0458x; 1.0088x over previous
import functools

import jax
import jax.numpy as jnp
from jax import lax
from jax.experimental import pallas as pl
from jax.experimental.pallas import tpu as pltpu
from jax.experimental.pallas import tpu_sc as plsc

_NUM_CODES = 8192
_EMBED_DIM = 1024
_M_BLK = 256
_C_BLK = 512
_NORM_BLK = 1024


def _norm_body(x_ref, out_ref, n_ref):
    x = x_ref[...]
    n = jnp.sqrt(jnp.sum(x * x, axis=1, keepdims=True))
    out_ref[...] = x / jnp.maximum(n, 1e-12)
    n_ref[0, 0, :] = n[:, 0]


def _normalize(x):
    m, d = x.shape
    nb = m // _NORM_BLK
    return pl.pallas_call(
        _norm_body,
        grid=(nb,),
        in_specs=[pl.BlockSpec((_NORM_BLK, d), lambda i: (i, 0))],
        out_specs=[
            pl.BlockSpec((_NORM_BLK, d), lambda i: (i, 0)),
            pl.BlockSpec((1, 1, _NORM_BLK), lambda i: (i, 0, 0)),
        ],
        out_shape=[
            jax.ShapeDtypeStruct((m, d), jnp.float32),
            jax.ShapeDtypeStruct((nb, 1, _NORM_BLK), jnp.float32),
        ],
    )(x)


def _argmax_body(xhat_ref, what_ref, idx_ref, m_ref, m_scr, a_scr):
    j = pl.program_id(1)
    nj = pl.num_programs(1)

    xhat = xhat_ref[...]
    what = what_ref[...]

    s = lax.dot_general(xhat, what, (((1,), (1,)), ((), ())),
                        preferred_element_type=jnp.float32)
    col_i = jax.lax.broadcasted_iota(jnp.int32, s.shape, 1)

    def seg_stats(mask):
        sm = jnp.where(mask, s, -jnp.inf)
        m = jnp.max(sm, axis=1, keepdims=True)
        arg = jnp.argmax(sm, axis=1)[:, None].astype(jnp.int32)
        return m, arg + j * _C_BLK

    def merge(m, a):
        better = m > m_scr[...]
        a_scr[...] = jnp.where(better, a, a_scr[...])
        m_scr[...] = jnp.where(better, m, m_scr[...])

    boundary = (j == 5) | (j == 10)
    off = jnp.where(j == 5, 176, jnp.where(j == 10, 352, _C_BLK))
    mA, aA = seg_stats(col_i < off)

    @pl.when(j == 0)
    def _():
        m_scr[...] = mA
        a_scr[...] = aA

    @pl.when(j > 0)
    def _():
        merge(mA, aA)

    @pl.when(boundary)
    def _():
        m_scr[...] = m_scr[...].astype(jnp.bfloat16).astype(jnp.float32)
        mB, aB = seg_stats(col_i >= off)
        merge(mB, aB)

    @pl.when(j == nj - 1)
    def _():
        idx_ref[0, 0, :] = a_scr[...][:, 0]
        m_ref[0, 0, :] = m_scr[...][:, 0]


def _matmul_argmax(xhat, what):
    m, d = xhat.shape
    c = what.shape[0]
    ni, nj = m // _M_BLK, c // _C_BLK
    return pl.pallas_call(
        _argmax_body,
        grid=(ni, nj),
        in_specs=[
            pl.BlockSpec((_M_BLK, d), lambda i, j: (i, 0)),
            pl.BlockSpec((_C_BLK, d), lambda i, j: (j, 0)),
        ],
        out_specs=[
            pl.BlockSpec((1, 1, _M_BLK), lambda i, j: (i, 0, 0)),
            pl.BlockSpec((1, 1, _M_BLK), lambda i, j: (i, 0, 0)),
        ],
        out_shape=[
            jax.ShapeDtypeStruct((ni, 1, _M_BLK), jnp.int32),
            jax.ShapeDtypeStruct((ni, 1, _M_BLK), jnp.float32),
        ],
        scratch_shapes=[
            pltpu.VMEM((_M_BLK, 1), jnp.float32),
            pltpu.VMEM((_M_BLK, 1), jnp.int32),
        ],
    )(xhat, what)


def _sc_gather_hist(W, wn, idx):
    n = idx.shape[0]
    d = W.shape[1]
    info = plsc.get_sparse_core_info()
    nw = info.num_cores * info.num_subcores
    per_w = n // nw
    chunk = 32
    mesh = plsc.VectorSubcoreMesh(core_axis_name="c", subcore_axis_name="s")

    @functools.partial(
        pl.kernel,
        out_type=(jax.ShapeDtypeStruct((n, d), jnp.float32),
                  jax.ShapeDtypeStruct((nw, _NUM_CODES), jnp.float32),
                  jax.ShapeDtypeStruct((n,), jnp.float32)),
        mesh=mesh,
        compiler_params=pltpu.CompilerParams(needs_layout_passes=False),
        scratch_types=[
            pltpu.VMEM((per_w,), jnp.int32),
            pltpu.VMEM((chunk, d), jnp.float32),
            pltpu.VMEM((_NUM_CODES,), jnp.float32),
            pltpu.VMEM((_NUM_CODES,), jnp.float32),
            pltpu.VMEM((per_w,), jnp.float32),
            pltpu.SemaphoreType.DMA,
        ],
    )
    def k(w_hbm, wn_hbm, idx_hbm, out_hbm, cnt_hbm, wns_hbm,
          idx_v, rows_v, cnt_v, wn_v, wns_v, sem):
        wid = lax.axis_index("s") * info.num_cores + lax.axis_index("c")
        base = wid * per_w
        pltpu.sync_copy(idx_hbm.at[pl.ds(base, per_w)], idx_v)
        pltpu.sync_copy(wn_hbm, wn_v)

        zeros16 = jnp.zeros((16,), jnp.float32)

        def zbody(t, carry):
            cnt_v[pl.ds(t * 16, 16)] = zeros16
            return carry

        lax.fori_loop(0, _NUM_CODES // 16, zbody, 0)

        ones16 = jnp.ones((16,), jnp.float32)

        def hbody(t, carry):
            v = idx_v[pl.ds(t * 16, 16)]
            plsc.addupdate_scatter(cnt_v, [v], ones16)
            wns_v[pl.ds(t * 16, 16)] = plsc.load_gather(wn_v, [v])
            return carry

        lax.fori_loop(0, per_w // 16, hbody, 0)
        pltpu.sync_copy(cnt_v, cnt_hbm.at[wid])
        pltpu.sync_copy(wns_v, wns_hbm.at[pl.ds(base, per_w)])

        def gbody(g, carry):
            pltpu.async_copy(w_hbm.at[idx_v.at[pl.ds(g * chunk, chunk)]],
                             rows_v, sem).wait()
            pltpu.sync_copy(rows_v, out_hbm.at[pl.ds(base + g * chunk, chunk)])
            return carry

        lax.fori_loop(0, per_w // chunk, gbody, 0)

    return k(W, wn, idx)


def _final_body(xn_ref, m_ref, wns_ref, cnt_ref, loss_ref, perp_ref):
    xn = xn_ref[...]
    m = m_ref[...]
    wns = wns_ref[...]
    lrow = xn * xn - 2.0 * (m * xn * wns) + wns * wns
    loss = jnp.sum(lrow) * (1.25 / (16384.0 * 1024.0))
    loss_ref[...] = jnp.full((1, 1), loss, jnp.float32)

    counts = jnp.sum(cnt_ref[...], axis=0, keepdims=True)
    probs = counts * (1.0 / 16384.0)
    ent = -jnp.sum(probs * jnp.log(probs + 1e-10))
    perp_ref[...] = jnp.full((1, 1), jnp.exp(ent), jnp.float32)


def _finalize(xn, m, wns, cnt):
    return pl.pallas_call(
        _final_body,
        out_shape=[
            jax.ShapeDtypeStruct((1, 1), jnp.float32),
            jax.ShapeDtypeStruct((1, 1), jnp.float32),
        ],
    )(xn, m, wns, cnt)


def kernel(inputs, W):
    b, npatch, d = inputs.shape
    n = b * npatch
    x = inputs.reshape(-1, d)
    xhat, xn3 = _normalize(x)
    what, wn3 = _normalize(W)
    idx3, m3 = _matmul_argmax(xhat, what)
    idx = idx3.reshape(-1)
    quant, cnt, wns = _sc_gather_hist(W, wn3.reshape(-1), idx)
    loss, perp = _finalize(xn3.reshape(1, n), m3.reshape(1, n),
                           wns.reshape(1, n), cnt)
    return (quant.reshape(b, npatch, d), loss.reshape(()), perp.reshape(()))


# speedup vs baseline: 1.1317x; 1.0821x over previous
import functools

import jax
import jax.numpy as jnp
from jax import lax
from jax.experimental import pallas as pl
from jax.experimental.pallas import tpu as pltpu
from jax.experimental.pallas import tpu_sc as plsc

_NUM_CODES = 8192
_EMBED_DIM = 1024
_M_BLK = 256
_C_BLK = 512
_NORM_BLK = 1024


def _norm_body(x_ref, out_ref, n_ref):
    x = x_ref[...]
    n = jnp.sqrt(jnp.sum(x * x, axis=1, keepdims=True))
    out_ref[...] = x / jnp.maximum(n, 1e-12)
    n_ref[0, 0, :] = n[:, 0]


def _normalize(x):
    m, d = x.shape
    nb = m // _NORM_BLK
    return pl.pallas_call(
        _norm_body,
        grid=(nb,),
        in_specs=[pl.BlockSpec((_NORM_BLK, d), lambda i: (i, 0))],
        out_specs=[
            pl.BlockSpec((_NORM_BLK, d), lambda i: (i, 0)),
            pl.BlockSpec((1, 1, _NORM_BLK), lambda i: (i, 0, 0)),
        ],
        out_shape=[
            jax.ShapeDtypeStruct((m, d), jnp.float32),
            jax.ShapeDtypeStruct((nb, 1, _NORM_BLK), jnp.float32),
        ],
    )(x)


def _argmax_body(xhat_ref, what_ref, idx_ref, m_ref, m_scr, a_scr):
    j = pl.program_id(1)
    nj = pl.num_programs(1)

    xhat = xhat_ref[...]
    what = what_ref[...]

    s = lax.dot_general(xhat, what, (((1,), (1,)), ((), ())),
                        preferred_element_type=jnp.float32)
    lanes = 128
    nch = _C_BLK // lanes
    lane_f = jax.lax.broadcasted_iota(jnp.int32, (_M_BLK, lanes), 1
                                      ).astype(jnp.float32)
    base_f = (j * _C_BLK).astype(jnp.float32)

    @pl.when(j == 0)
    def _():
        m_scr[...] = jnp.full((_M_BLK, lanes), -jnp.inf, jnp.float32)
        a_scr[...] = jnp.zeros((_M_BLK, lanes), jnp.float32)

    def scan_chunks(lo, hi):
        for c in range(nch):
            chunk = s[:, c * lanes:(c + 1) * lanes]
            col = lane_f + float(c * lanes)
            mask = (col >= lo) & (col < hi)
            cand = jnp.where(mask, chunk, -jnp.inf)
            better = cand > m_scr[...]
            a_scr[...] = jnp.where(better, col + base_f, a_scr[...])
            m_scr[...] = jnp.where(better, cand, m_scr[...])

    def resolve():
        mv = m_scr[...]
        mg = jnp.max(mv, axis=1, keepdims=True)
        cg = jnp.min(jnp.where(mv == mg, a_scr[...], 1e9),
                     axis=1, keepdims=True)
        return mg, cg

    boundary = (j == 5) | (j == 10)
    off = jnp.where(j == 5, 176.0, jnp.where(j == 10, 352.0,
                                             float(_C_BLK)))
    scan_chunks(jnp.float32(0.0), off)

    @pl.when(boundary)
    def _():
        mg, cg = resolve()
        mg = mg.astype(jnp.bfloat16).astype(jnp.float32)
        m_scr[...] = jnp.broadcast_to(mg, (_M_BLK, lanes))
        a_scr[...] = jnp.broadcast_to(cg, (_M_BLK, lanes))
        scan_chunks(off, jnp.float32(float(_C_BLK)))

    @pl.when(j == nj - 1)
    def _():
        mg, cg = resolve()
        idx_ref[0, 0, :] = cg[:, 0].astype(jnp.int32)
        m_ref[0, 0, :] = mg[:, 0]


def _matmul_argmax(xhat, what):
    m, d = xhat.shape
    c = what.shape[0]
    ni, nj = m // _M_BLK, c // _C_BLK
    return pl.pallas_call(
        _argmax_body,
        grid=(ni, nj),
        in_specs=[
            pl.BlockSpec((_M_BLK, d), lambda i, j: (i, 0)),
            pl.BlockSpec((_C_BLK, d), lambda i, j: (j, 0)),
        ],
        out_specs=[
            pl.BlockSpec((1, 1, _M_BLK), lambda i, j: (i, 0, 0)),
            pl.BlockSpec((1, 1, _M_BLK), lambda i, j: (i, 0, 0)),
        ],
        out_shape=[
            jax.ShapeDtypeStruct((ni, 1, _M_BLK), jnp.int32),
            jax.ShapeDtypeStruct((ni, 1, _M_BLK), jnp.float32),
        ],
        scratch_shapes=[
            pltpu.VMEM((_M_BLK, 128), jnp.float32),
            pltpu.VMEM((_M_BLK, 128), jnp.float32),
        ],
    )(xhat, what)


def _sc_gather_hist(W, wn, idx):
    n = idx.shape[0]
    d = W.shape[1]
    info = plsc.get_sparse_core_info()
    nw = info.num_cores * info.num_subcores
    per_w = n // nw
    chunk = 32
    mesh = plsc.VectorSubcoreMesh(core_axis_name="c", subcore_axis_name="s")

    @functools.partial(
        pl.kernel,
        out_type=(jax.ShapeDtypeStruct((n, d), jnp.float32),
                  jax.ShapeDtypeStruct((nw, _NUM_CODES), jnp.float32),
                  jax.ShapeDtypeStruct((n,), jnp.float32)),
        mesh=mesh,
        compiler_params=pltpu.CompilerParams(needs_layout_passes=False),
        scratch_types=[
            pltpu.VMEM((per_w,), jnp.int32),
            pltpu.VMEM((chunk, d), jnp.float32),
            pltpu.VMEM((_NUM_CODES,), jnp.float32),
            pltpu.VMEM((_NUM_CODES,), jnp.float32),
            pltpu.VMEM((per_w,), jnp.float32),
            pltpu.SemaphoreType.DMA,
        ],
    )
    def k(w_hbm, wn_hbm, idx_hbm, out_hbm, cnt_hbm, wns_hbm,
          idx_v, rows_v, cnt_v, wn_v, wns_v, sem):
        wid = lax.axis_index("s") * info.num_cores + lax.axis_index("c")
        base = wid * per_w
        pltpu.sync_copy(idx_hbm.at[pl.ds(base, per_w)], idx_v)
        pltpu.sync_copy(wn_hbm, wn_v)

        zeros16 = jnp.zeros((16,), jnp.float32)

        def zbody(t, carry):
            cnt_v[pl.ds(t * 16, 16)] = zeros16
            return carry

        lax.fori_loop(0, _NUM_CODES // 16, zbody, 0)

        ones16 = jnp.ones((16,), jnp.float32)

        def hbody(t, carry):
            v = idx_v[pl.ds(t * 16, 16)]
            plsc.addupdate_scatter(cnt_v, [v], ones16)
            wns_v[pl.ds(t * 16, 16)] = plsc.load_gather(wn_v, [v])
            return carry

        lax.fori_loop(0, per_w // 16, hbody, 0)
        pltpu.sync_copy(cnt_v, cnt_hbm.at[wid])
        pltpu.sync_copy(wns_v, wns_hbm.at[pl.ds(base, per_w)])

        def gbody(g, carry):
            pltpu.async_copy(w_hbm.at[idx_v.at[pl.ds(g * chunk, chunk)]],
                             rows_v, sem).wait()
            pltpu.sync_copy(rows_v, out_hbm.at[pl.ds(base + g * chunk, chunk)])
            return carry

        lax.fori_loop(0, per_w // chunk, gbody, 0)

    return k(W, wn, idx)


def _final_body(xn_ref, m_ref, wns_ref, cnt_ref, loss_ref, perp_ref):
    xn = xn_ref[...]
    m = m_ref[...]
    wns = wns_ref[...]
    lrow = xn * xn - 2.0 * (m * xn * wns) + wns * wns
    loss = jnp.sum(lrow) * (1.25 / (16384.0 * 1024.0))
    loss_ref[...] = jnp.full((1, 1), loss, jnp.float32)

    counts = jnp.sum(cnt_ref[...], axis=0, keepdims=True)
    probs = counts * (1.0 / 16384.0)
    ent = -jnp.sum(probs * jnp.log(probs + 1e-10))
    perp_ref[...] = jnp.full((1, 1), jnp.exp(ent), jnp.float32)


def _finalize(xn, m, wns, cnt):
    return pl.pallas_call(
        _final_body,
        out_shape=[
            jax.ShapeDtypeStruct((1, 1), jnp.float32),
            jax.ShapeDtypeStruct((1, 1), jnp.float32),
        ],
    )(xn, m, wns, cnt)


def kernel(inputs, W):
    b, npatch, d = inputs.shape
    n = b * npatch
    x = inputs.reshape(-1, d)
    xhat, xn3 = _normalize(x)
    what, wn3 = _normalize(W)
    idx3, m3 = _matmul_argmax(xhat, what)
    idx = idx3.reshape(-1)
    quant, cnt, wns = _sc_gather_hist(W, wn3.reshape(-1), idx)
    loss, perp = _finalize(xn3.reshape(1, n), m3.reshape(1, n),
                           wns.reshape(1, n), cnt)
    return (quant.reshape(b, npatch, d), loss.reshape(()), perp.reshape(()))
